# Initial kernel scaffold; baseline (speedup 1.0000x reference)
import jax
import jax.numpy as jnp
from jax.experimental import pallas as pl


def kernel(time, table):
    raise NotImplementedError("write your pallas kernel here")


# speedup vs baseline: 5.6508x; 5.6508x over previous
import functools

import jax
import jax.numpy as jnp
from jax import lax
from jax.experimental import pallas as pl
from jax.experimental.pallas import tpu as pltpu
from jax.experimental.pallas import tpu_sc as plsc


@functools.lru_cache(maxsize=None)
def _make_sc_gather(n_idx: int, n_rows: int, d: int):
    info = plsc.get_sparse_core_info()
    nc, ns = info.num_cores, info.num_subcores
    nw = nc * ns
    assert n_idx % nw == 0
    b_per_w = n_idx // nw
    chunk = 640
    nbuf = 3
    assert b_per_w % chunk == 0
    n_chunks = b_per_w // chunk
    mesh = plsc.VectorSubcoreMesh(core_axis_name="c", subcore_axis_name="s")

    @functools.partial(
        pl.kernel,
        mesh=mesh,
        out_type=jax.ShapeDtypeStruct((n_idx, d), jnp.float32),
        scratch_types=[
            pltpu.VMEM((nbuf, chunk), jnp.int32),
            pltpu.VMEM((nbuf, chunk, d), jnp.float32),
            pltpu.VMEM_SHARED((n_rows, d), jnp.float32),
            pltpu.SemaphoreType.DMA((nbuf,)),
            pltpu.SemaphoreType.DMA((nbuf,)),
        ],
        compiler_params=pltpu.CompilerParams(use_tc_tiling_on_sc=False),
    )
    def k(table_hbm, idx_hbm, out_hbm, idx_v, rows_v, tab_sh, sem_g, sem_o):
        cid = lax.axis_index("c")
        sid = lax.axis_index("s")
        wid = sid * nc + cid
        base = wid * b_per_w

        @pl.when(sid == 0)
        def _():
            pltpu.sync_copy(table_hbm, tab_sh)

        plsc.subcore_barrier()

        def start_gather(g, b):
            off = base + g * chunk
            pltpu.sync_copy(idx_hbm.at[pl.ds(off, chunk)], idx_v.at[b])
            pltpu.async_copy(tab_sh.at[idx_v.at[b]], rows_v.at[b], sem_g.at[b])

        def wait_gather(b):
            pltpu.make_async_copy(
                tab_sh.at[idx_v.at[b]], rows_v.at[b], sem_g.at[b]
            ).wait()

        def start_store(g, b):
            off = base + g * chunk
            pltpu.async_copy(rows_v.at[b], out_hbm.at[pl.ds(off, chunk)], sem_o.at[b])

        def wait_store(b):
            pltpu.make_async_copy(
                rows_v.at[b], out_hbm.at[pl.ds(0, chunk)], sem_o.at[b]
            ).wait()

        for b in range(nbuf):
            start_gather(b, b)

        def body(g, carry):
            b = lax.rem(g, nbuf)
            wait_gather(b)
            start_store(g, b)
            nxt = g + nbuf

            @pl.when(nxt < n_chunks)
            def _():
                wait_store(b)
                start_gather(nxt, b)

            return carry

        lax.fori_loop(0, n_chunks, body, 0)

        for b in range(nbuf):
            wait_store(b)

    return k


def kernel(time, table):
    n, t = time.shape
    d = table.shape[1]
    idx = time.reshape(n * t).astype(jnp.int32)
    out = _make_sc_gather(n * t, table.shape[0], d)(table, idx)
    return out.reshape(n, t, d)
